# baseline (speedup 1.0000x reference)
import functools

import jax
import jax.numpy as jnp
from jax import lax
from jax.experimental import pallas as pl
from jax.experimental.pallas import tpu as pltpu
from jax.experimental.pallas import tpu_sc as plsc

_N = 10000
_E = 320000
_H = 128
_G = 256
_NCONV = 5
_NFC = 3

_NC = 2
_NS = 16
_NW = _NC * _NS
_EPW = _E // _NW
_K = 80
_NCH = _EPW // _K
_NP = 10240
_RPT = _NP // _NS
_DW = 16


def _mesh():
    return plsc.VectorSubcoreMesh(core_axis_name="c", subcore_axis_name="s")


def _deg_partial(dst3):
    @functools.partial(
        pl.kernel,
        out_type=jax.ShapeDtypeStruct((_NC, _NP), jnp.float32),
        mesh=_mesh(),
        scratch_types=[
            pltpu.VMEM((_NCH, _K), jnp.int32),
            pltpu.VMEM((_K,), jnp.float32),
            pltpu.VMEM_SHARED((_NP,), jnp.float32),
        ],
    )
    def k(dst_hbm, out_hbm, dstv, vals, degsp):
        c = lax.axis_index("c")
        s = lax.axis_index("s")
        wid = s * _NC + c
        pltpu.sync_copy(dst_hbm.at[wid], dstv)

        def fill(v):
            def body(i, _):
                vals[pl.ds(i * 16, 16)] = jnp.full((16,), v, jnp.float32)
                return 0
            lax.fori_loop(0, _K // 16, body, 0)

        fill(0.0)
        for z in range(_RPT // _K):
            pltpu.sync_copy(vals, degsp.at[pl.ds(s * _RPT + z * _K, _K)])
        fill(1.0)
        plsc.subcore_barrier()

        def chunk(j, _):
            pltpu.sync_copy(vals, degsp.at[dstv.at[j]], add=True)
            return 0

        lax.fori_loop(0, _NCH, chunk, 0)
        plsc.subcore_barrier()
        pltpu.sync_copy(degsp.at[pl.ds(s * _RPT, _RPT)],
                        out_hbm.at[c, pl.ds(s * _RPT, _RPT)])

    return k(dst3)


def _edge_scatter(hw, src3, dst3):
    @functools.partial(
        pl.kernel,
        out_type=jax.ShapeDtypeStruct((_NC, _NP, _H), jnp.float32),
        mesh=_mesh(),
        scratch_types=[
            pltpu.VMEM((_EPW,), jnp.int32),
            pltpu.VMEM((_NCH, _K), jnp.int32),
            pltpu.VMEM((_K, _H), jnp.float32),
            pltpu.VMEM((_K, _H), jnp.float32),
            pltpu.VMEM_SHARED((_NP, _H), jnp.float32),
            pltpu.SemaphoreType.DMA,
            pltpu.SemaphoreType.DMA,
            pltpu.SemaphoreType.DMA,
            pltpu.SemaphoreType.DMA,
        ],
    )
    def k(hw_hbm, src_hbm, dst_hbm, out_hbm, srcv, dstv, rows_a, rows_b,
          aggsp, sem_a, sem_b, sem_sa, sem_sb):
        c = lax.axis_index("c")
        s = lax.axis_index("s")
        wid = s * _NC + c
        pltpu.sync_copy(src_hbm.at[wid], srcv)
        pltpu.sync_copy(dst_hbm.at[wid], dstv)

        def fill_zero(i, _):
            for j in range(_H // 16):
                rows_a[i, pl.ds(j * 16, 16)] = jnp.zeros((16,), jnp.float32)
            return 0

        lax.fori_loop(0, _K, fill_zero, 0)
        for z in range(_RPT // _K):
            pltpu.sync_copy(rows_a, aggsp.at[pl.ds(s * _RPT + z * _K, _K)])
        plsc.subcore_barrier()

        def gather(j, buf, sem):
            pltpu.async_copy(hw_hbm.at[srcv.at[pl.ds(j * _K, _K)]], buf, sem)

        def drain(buf, sem):
            pltpu.make_async_copy(hw_hbm.at[pl.ds(0, _K)], buf, sem).wait()

        def scatter(j, buf, sem):
            pltpu.async_copy(buf, aggsp.at[dstv.at[j]], sem, add=True)

        gather(0, rows_a, sem_a)
        gather(1, rows_b, sem_b)

        def pair(i, _):
            j0 = 2 * i
            drain(rows_a, sem_a)
            scatter(j0, rows_a, sem_sa)
            drain(rows_b, sem_b)
            scatter(j0 + 1, rows_b, sem_sb)
            drain(rows_a, sem_sa)

            @pl.when(j0 + 2 < _NCH)
            def _():
                gather(j0 + 2, rows_a, sem_a)

            drain(rows_b, sem_sb)

            @pl.when(j0 + 3 < _NCH)
            def _():
                gather(j0 + 3, rows_b, sem_b)

            return 0

        lax.fori_loop(0, _NCH // 2, pair, 0)
        drain(rows_a, sem_a)
        scatter(_NCH - 1, rows_a, sem_sa)
        drain(rows_a, sem_sa)
        plsc.subcore_barrier()
        pltpu.sync_copy(aggsp.at[pl.ds(s * _RPT, _RPT)],
                        out_hbm.at[c, pl.ds(s * _RPT, _RPT)])

    return k(hw, src3, dst3)


def _dinv(degp_ref):
    deg = degp_ref[:, 0:1] + degp_ref[:, 1:2] + 1.0
    return lax.rsqrt(deg)


def _bn_relu(t, gamma, beta):
    mu = jnp.mean(t, axis=0, keepdims=True)
    var = jnp.mean((t - mu) ** 2, axis=0, keepdims=True)
    return jnp.maximum(gamma * (t - mu) / jnp.sqrt(var + 1e-5) + beta, 0.0)


def _emb_body(x_ref, ew_ref, eb_ref, w0_ref, degp_ref, out_ref):
    h = jnp.dot(x_ref[...], ew_ref[...], preferred_element_type=jnp.float32)
    h = h + eb_ref[...]
    h = jnp.where(h > 0, h, jnp.exp(jnp.minimum(h, 0.0)) - 1.0)
    out_ref[...] = _dinv(degp_ref) * jnp.dot(
        h, w0_ref[...], preferred_element_type=jnp.float32)


def _conv_body(p_ref, hwp_ref, degp_ref, b_ref, g_ref, be_ref, wn_ref, out_ref):
    dinv = _dinv(degp_ref)
    t = dinv * (p_ref[0][:_N] + p_ref[1][:_N] + hwp_ref[...]) + b_ref[...]
    h = _bn_relu(t, g_ref[...], be_ref[...])
    out_ref[...] = dinv * jnp.dot(h, wn_ref[...],
                                  preferred_element_type=jnp.float32)


def _conv_last_body(p_ref, hwp_ref, degp_ref, b_ref, g_ref, be_ref, out_ref):
    dinv = _dinv(degp_ref)
    t = dinv * (p_ref[0][:_N] + p_ref[1][:_N] + hwp_ref[...]) + b_ref[...]
    out_ref[...] = _bn_relu(t, g_ref[...], be_ref[...])


def _fc_body(h_ref, batch_ref, fw_ref, fb_ref, fg_ref, fbe_ref, ow_ref,
             ob_ref, out_ref):
    bi = jnp.broadcast_to(batch_ref[...], (_G, _N))
    gid = lax.broadcasted_iota(jnp.int32, (_G, _N), 0)
    oh = (bi == gid).astype(jnp.float32)
    g = jnp.dot(oh, h_ref[...], preferred_element_type=jnp.float32)
    for i in range(_NFC):
        t = jnp.dot(g, fw_ref[i], preferred_element_type=jnp.float32)
        t = t + fb_ref[i]
        g = _bn_relu(t, fg_ref[i], fbe_ref[i])
    logits = jnp.dot(g, ow_ref[...], preferred_element_type=jnp.float32)
    logits = logits + ob_ref[...]
    col = lax.broadcasted_iota(jnp.int32, (_G, _H), 1)
    valid = col < 2
    ml = jnp.max(jnp.where(valid, logits, -1e30), axis=1, keepdims=True)
    e = jnp.where(valid, jnp.exp(logits - ml), 0.0)
    lse = jnp.log(jnp.sum(e, axis=1, keepdims=True)) + ml
    out_ref[...] = logits - lse


def _tc(body, out_shape, *args):
    return pl.pallas_call(
        body, out_shape=jax.ShapeDtypeStruct(out_shape, jnp.float32))(*args)


def kernel(x, edge_index, batch, emb_W, emb_b, conv_W, conv_b, conv_gamma,
           conv_beta, fc_W, fc_b, fc_gamma, fc_beta, out_W, out_b):
    src3 = edge_index[0].reshape(_NW, _EPW)
    dst3 = edge_index[1].reshape(_NW, _NCH, _K)

    degp = _deg_partial(dst3)[:, :_N].T

    emb_b2 = emb_b.reshape(1, _H)
    hw = _tc(_emb_body, (_N, _H), x, emb_W, emb_b2, conv_W[0], degp)

    for i in range(_NCONV):
        p = _edge_scatter(hw, src3, dst3)
        b2 = conv_b[i].reshape(1, _H)
        g2 = conv_gamma[i].reshape(1, _H)
        be2 = conv_beta[i].reshape(1, _H)
        if i + 1 < _NCONV:
            hw = _tc(_conv_body, (_N, _H), p, hw, degp, b2, g2, be2,
                     conv_W[i + 1])
        else:
            h5 = _tc(_conv_last_body, (_N, _H), p, hw, degp, b2, g2, be2)

    batch2 = batch.reshape(1, _N)
    fb3 = fc_b.reshape(_NFC, 1, _H)
    fg3 = fc_gamma.reshape(_NFC, 1, _H)
    fbe3 = fc_beta.reshape(_NFC, 1, _H)
    ow = jnp.pad(out_W, ((0, 0), (0, _H - 2)))
    ob = jnp.pad(out_b, (0, _H - 2)).reshape(1, _H)
    out = _tc(_fc_body, (_G, _H), h5, batch2, fc_W, fb3, fg3, fbe3, ow, ob)
    return out[:, :2]


# speedup vs baseline: 1.2471x; 1.2471x over previous
import functools

import jax
import jax.numpy as jnp
from jax import lax
from jax.experimental import pallas as pl
from jax.experimental.pallas import tpu as pltpu
from jax.experimental.pallas import tpu_sc as plsc

_N = 10000
_E = 320000
_H = 128
_G = 256
_NCONV = 5
_NFC = 3

_NC = 2
_NS = 16
_NW = _NC * _NS
_EPW = _E // _NW
_K = 80
_NCH = _EPW // _K
_NP = 10240
_RPT = _NP // _NS
_DW = 16


def _mesh():
    return plsc.VectorSubcoreMesh(core_axis_name="c", subcore_axis_name="s")


def _deg_partial(dst3):
    @functools.partial(
        pl.kernel,
        out_type=jax.ShapeDtypeStruct((_NC, _NP), jnp.float32),
        mesh=_mesh(),
        scratch_types=[
            pltpu.VMEM((_NCH, _K), jnp.int32),
            pltpu.VMEM((_K,), jnp.float32),
            pltpu.VMEM_SHARED((_NP,), jnp.float32),
        ],
    )
    def k(dst_hbm, out_hbm, dstv, vals, degsp):
        c = lax.axis_index("c")
        s = lax.axis_index("s")
        wid = s * _NC + c
        pltpu.sync_copy(dst_hbm.at[wid], dstv)

        def fill(v):
            def body(i, _):
                vals[pl.ds(i * 16, 16)] = jnp.full((16,), v, jnp.float32)
                return 0
            lax.fori_loop(0, _K // 16, body, 0)

        fill(0.0)
        for z in range(_RPT // _K):
            pltpu.sync_copy(vals, degsp.at[pl.ds(s * _RPT + z * _K, _K)])
        fill(1.0)
        plsc.subcore_barrier()

        def chunk(j, _):
            pltpu.sync_copy(vals, degsp.at[dstv.at[j]], add=True)
            return 0

        lax.fori_loop(0, _NCH, chunk, 0)
        plsc.subcore_barrier()
        pltpu.sync_copy(degsp.at[pl.ds(s * _RPT, _RPT)],
                        out_hbm.at[c, pl.ds(s * _RPT, _RPT)])

    return k(dst3)


def _edge_scatter(hw, src3, dst3):
    @functools.partial(
        pl.kernel,
        out_type=jax.ShapeDtypeStruct((_NC, _NP, _H), jnp.float32),
        mesh=_mesh(),
        scratch_types=[
            pltpu.VMEM((_EPW,), jnp.int32),
            pltpu.VMEM((_NCH, _K), jnp.int32),
            pltpu.VMEM((_K, _H), jnp.float32),
            pltpu.VMEM((_K, _H), jnp.float32),
            pltpu.VMEM_SHARED((_NP, _H), jnp.float32),
            pltpu.SemaphoreType.DMA,
            pltpu.SemaphoreType.DMA,
        ],
    )
    def k(hw_hbm, src_hbm, dst_hbm, out_hbm, srcv, dstv, rows_a, rows_b,
          aggsp, sem_a, sem_b):
        c = lax.axis_index("c")
        s = lax.axis_index("s")
        wid = s * _NC + c
        pltpu.sync_copy(src_hbm.at[wid], srcv)
        pltpu.sync_copy(dst_hbm.at[wid], dstv)

        def fill_zero(i, _):
            for j in range(_H // 16):
                rows_a[i, pl.ds(j * 16, 16)] = jnp.zeros((16,), jnp.float32)
            return 0

        lax.fori_loop(0, _K, fill_zero, 0)
        for z in range(_RPT // _K):
            pltpu.sync_copy(rows_a, aggsp.at[pl.ds(s * _RPT + z * _K, _K)])
        plsc.subcore_barrier()

        def gather(j, buf, sem):
            return pltpu.async_copy(
                hw_hbm.at[srcv.at[pl.ds(j * _K, _K)]], buf, sem)

        def drain(buf, sem):
            pltpu.make_async_copy(hw_hbm.at[pl.ds(0, _K)], buf, sem).wait()

        def scatter(j, buf):
            pltpu.sync_copy(buf, aggsp.at[dstv.at[j]], add=True)

        gather(0, rows_a, sem_a)

        def pair(i, _):
            j0 = 2 * i
            gather(j0 + 1, rows_b, sem_b)
            drain(rows_a, sem_a)
            scatter(j0, rows_a)
            gather(j0 + 2, rows_a, sem_a)
            drain(rows_b, sem_b)
            scatter(j0 + 1, rows_b)
            return 0

        lax.fori_loop(0, (_NCH - 1) // 2, pair, 0)
        drain(rows_a, sem_a)
        scatter(_NCH - 1, rows_a)
        plsc.subcore_barrier()
        pltpu.sync_copy(aggsp.at[pl.ds(s * _RPT, _RPT)],
                        out_hbm.at[c, pl.ds(s * _RPT, _RPT)])

    return k(hw, src3, dst3)


def _dinv(degp_ref):
    deg = degp_ref[:, 0:1] + degp_ref[:, 1:2] + 1.0
    return lax.rsqrt(deg)


def _bn_relu(t, gamma, beta):
    mu = jnp.mean(t, axis=0, keepdims=True)
    var = jnp.mean((t - mu) ** 2, axis=0, keepdims=True)
    return jnp.maximum(gamma * (t - mu) / jnp.sqrt(var + 1e-5) + beta, 0.0)


def _emb_body(x_ref, ew_ref, eb_ref, w0_ref, degp_ref, out_ref):
    h = jnp.dot(x_ref[...], ew_ref[...], preferred_element_type=jnp.float32)
    h = h + eb_ref[...]
    h = jnp.where(h > 0, h, jnp.exp(jnp.minimum(h, 0.0)) - 1.0)
    out_ref[...] = _dinv(degp_ref) * jnp.dot(
        h, w0_ref[...], preferred_element_type=jnp.float32)


def _conv_body(p_ref, hwp_ref, degp_ref, b_ref, g_ref, be_ref, wn_ref, out_ref):
    dinv = _dinv(degp_ref)
    t = dinv * (p_ref[0][:_N] + p_ref[1][:_N] + hwp_ref[...]) + b_ref[...]
    h = _bn_relu(t, g_ref[...], be_ref[...])
    out_ref[...] = dinv * jnp.dot(h, wn_ref[...],
                                  preferred_element_type=jnp.float32)


def _conv_last_body(p_ref, hwp_ref, degp_ref, b_ref, g_ref, be_ref, out_ref):
    dinv = _dinv(degp_ref)
    t = dinv * (p_ref[0][:_N] + p_ref[1][:_N] + hwp_ref[...]) + b_ref[...]
    out_ref[...] = _bn_relu(t, g_ref[...], be_ref[...])


def _fc_body(h_ref, batch_ref, fw_ref, fb_ref, fg_ref, fbe_ref, ow_ref,
             ob_ref, out_ref):
    bi = jnp.broadcast_to(batch_ref[...], (_G, _N))
    gid = lax.broadcasted_iota(jnp.int32, (_G, _N), 0)
    oh = (bi == gid).astype(jnp.float32)
    g = jnp.dot(oh, h_ref[...], preferred_element_type=jnp.float32)
    for i in range(_NFC):
        t = jnp.dot(g, fw_ref[i], preferred_element_type=jnp.float32)
        t = t + fb_ref[i]
        g = _bn_relu(t, fg_ref[i], fbe_ref[i])
    logits = jnp.dot(g, ow_ref[...], preferred_element_type=jnp.float32)
    logits = logits + ob_ref[...]
    col = lax.broadcasted_iota(jnp.int32, (_G, _H), 1)
    valid = col < 2
    ml = jnp.max(jnp.where(valid, logits, -1e30), axis=1, keepdims=True)
    e = jnp.where(valid, jnp.exp(logits - ml), 0.0)
    lse = jnp.log(jnp.sum(e, axis=1, keepdims=True)) + ml
    out_ref[...] = logits - lse


def _tc(body, out_shape, *args):
    return pl.pallas_call(
        body, out_shape=jax.ShapeDtypeStruct(out_shape, jnp.float32))(*args)


def kernel(x, edge_index, batch, emb_W, emb_b, conv_W, conv_b, conv_gamma,
           conv_beta, fc_W, fc_b, fc_gamma, fc_beta, out_W, out_b):
    src3 = edge_index[0].reshape(_NW, _EPW)
    dst3 = edge_index[1].reshape(_NW, _NCH, _K)

    degp = _deg_partial(dst3)[:, :_N].T

    emb_b2 = emb_b.reshape(1, _H)
    hw = _tc(_emb_body, (_N, _H), x, emb_W, emb_b2, conv_W[0], degp)

    for i in range(_NCONV):
        p = _edge_scatter(hw, src3, dst3)
        b2 = conv_b[i].reshape(1, _H)
        g2 = conv_gamma[i].reshape(1, _H)
        be2 = conv_beta[i].reshape(1, _H)
        if i + 1 < _NCONV:
            hw = _tc(_conv_body, (_N, _H), p, hw, degp, b2, g2, be2,
                     conv_W[i + 1])
        else:
            h5 = _tc(_conv_last_body, (_N, _H), p, hw, degp, b2, g2, be2)

    batch2 = batch.reshape(1, _N)
    fb3 = fc_b.reshape(_NFC, 1, _H)
    fg3 = fc_gamma.reshape(_NFC, 1, _H)
    fbe3 = fc_beta.reshape(_NFC, 1, _H)
    ow = jnp.pad(out_W, ((0, 0), (0, _H - 2)))
    ob = jnp.pad(out_b, (0, _H - 2)).reshape(1, _H)
    out = _tc(_fc_body, (_G, _H), h5, batch2, fc_W, fb3, fg3, fbe3, ow, ob)
    return out[:, :2]
